# Initial kernel scaffold; baseline (speedup 1.0000x reference)
import jax
import jax.numpy as jnp
from jax.experimental import pallas as pl


def kernel(x, edge_index, W1, b1, W2, b2, Wc, bc):
    raise NotImplementedError("write your pallas kernel here")


# speedup vs baseline: 31.1833x; 31.1833x over previous
import functools

import jax
import jax.numpy as jnp
from jax import lax
from jax.experimental import pallas as pl
from jax.experimental.pallas import tpu as pltpu
from jax.experimental.pallas import tpu_sc as plsc

N = 100000
E = 1600000
D_IN = 128
D_H = 32
D_OUT = 20

GROUP = 128
NROWS = E // GROUP
NSUB = 16
STRIPE = 6256
STRIPE_LAST = N - (NSUB - 1) * STRIPE
ROWS_PER_SUB = -(-NROWS // NSUB)
DEG_ROWS = NROWS // 2
DEG_PER_SUB = -(-DEG_ROWS // NSUB)

BN = 2000
NBLK = N // BN

def _stripe_copy(s, src_fn, dst_fn):
    @pl.when(s < NSUB - 1)
    def _():
        base = pl.multiple_of(s * STRIPE, 8)
        pltpu.sync_copy(src_fn(base, STRIPE), dst_fn(base, STRIPE))

    @pl.when(s == NSUB - 1)
    def _():
        base = (NSUB - 1) * STRIPE
        pltpu.sync_copy(src_fn(base, STRIPE_LAST), dst_fn(base, STRIPE_LAST))


def _deg_body(dst3d, ones_g, dega, degb, idx_v, ones_v, accv, tmpv,
              isem, ssem, part):
    c = lax.axis_index("c")
    s = lax.axis_index("s")
    pltpu.sync_copy(ones_g, ones_v)

    def zfill(k, _):
        accv[pl.ds(16 * k, 16)] = jnp.zeros((16,), jnp.float32)
        return _

    lax.fori_loop(0, STRIPE // 16, zfill, None)
    base_own = pl.multiple_of(s * N, 8)
    for k in range(NSUB - 1):
        pltpu.sync_copy(accv, part.at[pl.ds(base_own + k * STRIPE, STRIPE)])
    pltpu.sync_copy(accv.at[pl.ds(0, STRIPE_LAST)],
                    part.at[pl.ds(base_own + (NSUB - 1) * STRIPE,
                                  STRIPE_LAST)])
    plsc.subcore_barrier()

    def batch(bi, _):
        rs = [s + NSUB * (bi * NBUF + b) for b in range(NBUF)]
        full = [c * DEG_ROWS + r for r in rs]
        cps = [None] * NBUF
        for b in range(NBUF):
            @pl.when(rs[b] < DEG_ROWS)
            def _(b=b):
                cps[b] = pltpu.async_copy(
                    dst3d.at[full[b]], idx_v.at[b], isem.at[b])
        sca = [None] * NBUF
        for b in range(NBUF):
            @pl.when(rs[b] < DEG_ROWS)
            def _(b=b):
                cps[b].wait()

                def shift(k, _):
                    idx_v[b, 0, pl.ds(16 * k, 16)] = (
                        idx_v[b, 0, pl.ds(16 * k, 16)] + s * N)
                    return _

                lax.fori_loop(0, GROUP // 16, shift, None)
                sca[b] = pltpu.async_copy(
                    ones_v, part.at[idx_v.at[b].at[0]], ssem.at[b],
                    add=True)
        for b in range(NBUF):
            @pl.when(rs[b] < DEG_ROWS)
            def _(b=b):
                sca[b].wait()
        return _

    lax.fori_loop(0, -(-DEG_PER_SUB // NBUF), batch, None)
    plsc.subcore_barrier()

    def reduce_stripe(base, size):
        pltpu.sync_copy(part.at[pl.ds(base, size)], accv.at[pl.ds(0, size)])

        def add_part(j, _):
            off = pl.multiple_of(j * N + base, 8)
            pltpu.sync_copy(part.at[pl.ds(off, size)],
                            tmpv.at[pl.ds(0, size)])

            def vadd(k, _):
                accv[pl.ds(16 * k, 16)] = (accv[pl.ds(16 * k, 16)]
                                           + tmpv[pl.ds(16 * k, 16)])
                return _

            lax.fori_loop(0, size // 16, vadd, None)
            return _

        lax.fori_loop(1, NSUB, add_part, None)

    out = [dega, degb]

    @pl.when(s < NSUB - 1)
    def _():
        base = pl.multiple_of(s * STRIPE, 8)
        reduce_stripe(base, STRIPE)

    @pl.when(s == NSUB - 1)
    def _():
        reduce_stripe((NSUB - 1) * STRIPE, STRIPE_LAST)

    for cc in range(2):
        @pl.when(c == cc)
        def _(cc=cc):
            @pl.when(s < NSUB - 1)
            def _():
                base = pl.multiple_of(s * STRIPE, 8)
                pltpu.sync_copy(accv.at[pl.ds(0, STRIPE)],
                                out[cc].at[pl.ds(base, STRIPE)])

            @pl.when(s == NSUB - 1)
            def _():
                pltpu.sync_copy(
                    accv.at[pl.ds(0, STRIPE_LAST)],
                    out[cc].at[pl.ds((NSUB - 1) * STRIPE, STRIPE_LAST)])


NBUF = 8
AGG_BATCHES = -(-ROWS_PER_SUB // NBUF)


def _agg_body(g0, g1, src3d, dst3d, zeros_s, out0, out1,
              src_v, dst_v, rows_v, isem, gsem, ssem, acc):
    c = lax.axis_index("c")
    s = lax.axis_index("s")
    _stripe_copy(s, lambda b, n: zeros_s.at[pl.ds(0, n)],
                 lambda b, n: acc.at[pl.ds(b, n)])
    plsc.subcore_barrier()

    def edges(g_hbm):
        def batch(bi, _):
            rows = [s + NSUB * (bi * NBUF + b) for b in range(NBUF)]
            cps = [None] * NBUF
            for b in range(NBUF):
                @pl.when(rows[b] < NROWS)
                def _(b=b):
                    cps[b] = (
                        pltpu.async_copy(src3d.at[rows[b]], src_v.at[b],
                                         isem.at[b]),
                        pltpu.async_copy(dst3d.at[rows[b]], dst_v.at[b],
                                         isem.at[b]),
                    )
            gat = [None] * NBUF
            for b in range(NBUF):
                @pl.when(rows[b] < NROWS)
                def _(b=b):
                    cps[b][0].wait()
                    cps[b][1].wait()
                    gat[b] = pltpu.async_copy(
                        g_hbm.at[src_v.at[b].at[0]], rows_v.at[b],
                        gsem.at[b])
            sca = [None] * NBUF
            for b in range(NBUF):
                @pl.when(rows[b] < NROWS)
                def _(b=b):
                    gat[b].wait()
                    sca[b] = pltpu.async_copy(
                        rows_v.at[b], acc.at[dst_v.at[b].at[0]],
                        ssem.at[b], add=True)
            for b in range(NBUF):
                @pl.when(rows[b] < NROWS)
                def _(b=b):
                    sca[b].wait()
            return _

        lax.fori_loop(0, AGG_BATCHES, batch, None)

    @pl.when(c == 0)
    def _():
        edges(g0)

    @pl.when(c == 1)
    def _():
        edges(g1)

    plsc.subcore_barrier()

    @pl.when(c == 0)
    def _():
        _stripe_copy(s, lambda b, n: acc.at[pl.ds(b, n)],
                     lambda b, n: out0.at[pl.ds(b, n)])

    @pl.when(c == 1)
    def _():
        _stripe_copy(s, lambda b, n: acc.at[pl.ds(b, n)],
                     lambda b, n: out1.at[pl.ds(b, n)])


@functools.lru_cache(maxsize=1)
def _sc_kernels():
    mesh = plsc.VectorSubcoreMesh(core_axis_name="c", subcore_axis_name="s")
    params = pltpu.CompilerParams(use_tc_tiling_on_sc=False)
    deg_k = pl.kernel(
        _deg_body,
        mesh=mesh,
        compiler_params=params,
        out_type=[
            jax.ShapeDtypeStruct((N,), jnp.float32),
            jax.ShapeDtypeStruct((N,), jnp.float32),
        ],
        scratch_types=[
            pltpu.VMEM((NBUF, 1, GROUP), jnp.int32),
            pltpu.VMEM((GROUP,), jnp.float32),
            pltpu.VMEM((STRIPE,), jnp.float32),
            pltpu.VMEM((STRIPE,), jnp.float32),
            pltpu.SemaphoreType.DMA((NBUF,)),
            pltpu.SemaphoreType.DMA((NBUF,)),
            pltpu.VMEM_SHARED((NSUB * N,), jnp.float32),
        ],
    )
    agg_k = pl.kernel(
        _agg_body,
        mesh=mesh,
        compiler_params=params,
        out_type=[
            jax.ShapeDtypeStruct((N, 16), jnp.float32),
            jax.ShapeDtypeStruct((N, 16), jnp.float32),
        ],
        scratch_types=[
            pltpu.VMEM((NBUF, 1, GROUP), jnp.int32),
            pltpu.VMEM((NBUF, 1, GROUP), jnp.int32),
            pltpu.VMEM((NBUF, GROUP, 16), jnp.float32),
            pltpu.SemaphoreType.DMA((NBUF,)),
            pltpu.SemaphoreType.DMA((NBUF,)),
            pltpu.SemaphoreType.DMA((NBUF,)),
            pltpu.VMEM_SHARED((N, 16), jnp.float32),
        ],
    )
    return deg_k, agg_k


def _tc_dis_body(dega_ref, degb_ref, dis_ref):
    deg = dega_ref[...] + degb_ref[...]
    dis_ref[...] = jnp.where(deg > 0.0, lax.rsqrt(deg), 0.0)


def _tc1a_body(x_ref, w1_ref, h_ref):
    h_ref[...] = jnp.dot(x_ref[...], w1_ref[...],
                         preferred_element_type=jnp.float32)


def _tc1b_body(h_ref, dis_ref, g0_ref, g1_ref):
    g = h_ref[...] * dis_ref[...]
    g0_ref[...] = g[:, :16]
    g1_ref[...] = g[:, 16:]


def _tc2_body(a0_ref, a1_ref, dis_ref, b1_ref, w2_ref,
              g0_ref, g1_ref):
    dis = dis_ref[...]
    acc = jnp.concatenate([a0_ref[...], a1_ref[...]], axis=1)
    z = jnp.tanh(acc * dis + b1_ref[...])
    h2 = jnp.dot(z, w2_ref[...], preferred_element_type=jnp.float32)
    g = h2 * dis
    g0_ref[...] = g[:, :16]
    g1_ref[...] = g[:, 16:]


def _tc3_body(a0_ref, a1_ref, dis_ref, b2_ref, wc_ref, bc_ref,
              h_ref, out_ref):
    dis = dis_ref[...]
    acc = jnp.concatenate([a0_ref[...], a1_ref[...]], axis=1)
    z = jnp.tanh(acc * dis + b2_ref[...])
    h_ref[...] = z
    logits = jnp.dot(z, wc_ref[...], preferred_element_type=jnp.float32)
    logits = logits + bc_ref[...]
    m = jnp.max(logits, axis=1, keepdims=True)
    lse = m + jnp.log(jnp.sum(jnp.exp(logits - m), axis=1, keepdims=True))
    out_ref[...] = logits - lse


def _col_spec(d):
    return pl.BlockSpec((BN, d), lambda i: (i, 0))


def _full_spec(shape):
    nd = len(shape)
    return pl.BlockSpec(shape, lambda i: (0,) * nd)


def kernel(x, edge_index, W1, b1, W2, b2, Wc, bc):
    src = edge_index[0].astype(jnp.int32).reshape(NROWS, 1, GROUP)
    dst = edge_index[1].astype(jnp.int32).reshape(NROWS, 1, GROUP)
    ones_g = jnp.ones((GROUP,), jnp.float32)
    zeros_s = jnp.zeros((STRIPE, 16), jnp.float32)
    b1r = b1.reshape(1, D_H)
    b2r = b2.reshape(1, D_H)
    bcr = bc.reshape(1, D_OUT)

    deg_kernel, agg_kernel = _sc_kernels()

    tc1a = pl.pallas_call(
        _tc1a_body,
        grid=(NBLK,),
        in_specs=[_col_spec(D_IN), _full_spec((D_IN, D_H))],
        out_specs=_col_spec(D_H),
        out_shape=jax.ShapeDtypeStruct((N, D_H), jnp.float32),
    )
    h1 = tc1a(x, W1)

    dega, degb = deg_kernel(dst, ones_g)
    tc_dis = pl.pallas_call(
        _tc_dis_body,
        grid=(-(-N // 4096),),
        in_specs=[pl.BlockSpec((4096,), lambda i: (i,)),
                  pl.BlockSpec((4096,), lambda i: (i,))],
        out_specs=pl.BlockSpec((4096,), lambda i: (i,)),
        out_shape=jax.ShapeDtypeStruct((N,), jnp.float32),
    )
    dis = tc_dis(dega, degb).reshape(N, 1)

    tc1b = pl.pallas_call(
        _tc1b_body,
        grid=(NBLK,),
        in_specs=[_col_spec(D_H), _col_spec(1)],
        out_specs=[_col_spec(16), _col_spec(16)],
        out_shape=[jax.ShapeDtypeStruct((N, 16), jnp.float32),
                   jax.ShapeDtypeStruct((N, 16), jnp.float32)],
    )
    g0, g1 = tc1b(h1, dis)

    a0, a1 = agg_kernel(g0, g1, src, dst, zeros_s)

    tc2 = pl.pallas_call(
        _tc2_body,
        grid=(NBLK,),
        in_specs=[_col_spec(16), _col_spec(16), _col_spec(1),
                  _full_spec((1, D_H)), _full_spec((D_H, D_H))],
        out_specs=[_col_spec(16), _col_spec(16)],
        out_shape=[jax.ShapeDtypeStruct((N, 16), jnp.float32),
                   jax.ShapeDtypeStruct((N, 16), jnp.float32)],
    )
    g0b, g1b = tc2(a0, a1, dis, b1r, W2)

    c0, c1 = agg_kernel(g0b, g1b, src, dst, zeros_s)

    tc3 = pl.pallas_call(
        _tc3_body,
        grid=(NBLK,),
        in_specs=[_col_spec(16), _col_spec(16), _col_spec(1),
                  _full_spec((1, D_H)), _full_spec((D_H, D_OUT)),
                  _full_spec((1, D_OUT))],
        out_specs=[_col_spec(D_H), _col_spec(D_OUT)],
        out_shape=[jax.ShapeDtypeStruct((N, D_H), jnp.float32),
                   jax.ShapeDtypeStruct((N, D_OUT), jnp.float32)],
    )
    h, out = tc3(c0, c1, dis, b2r, Wc, bcr)
    return (out, h)
